# baseline (speedup 1.0000x reference)
import functools

import jax
import jax.numpy as jnp
from jax import lax
from jax.experimental import pallas as pl
from jax.experimental.pallas import tpu as pltpu
from jax.experimental.pallas import tpu_sc as plsc


def _make_gather(n_workers: int, per_w: int, chunk: int, n_ch: int,
                 n_total: int, d: int):
    mesh = plsc.VectorSubcoreMesh(core_axis_name="c", subcore_axis_name="s")

    @functools.partial(
        pl.kernel,
        mesh=mesh,
        out_type=jax.ShapeDtypeStruct((n_total, d), jnp.float32),
        scratch_types=[
            pltpu.VMEM((n_ch, chunk), jnp.int32),
            pltpu.VMEM((chunk, d), jnp.float32),
            pltpu.SemaphoreType.DMA,
        ],
        compiler_params=pltpu.CompilerParams(use_tc_tiling_on_sc=False),
    )
    def gather_kernel(table_hbm, idx_hbm, out_hbm, idx_v, buf, gsem):
        wid = lax.axis_index("s") * 2 + lax.axis_index("c")
        base = wid * per_w
        pltpu.sync_copy(idx_hbm.at[wid], idx_v)

        def body(j, carry):
            pltpu.async_copy(table_hbm.at[idx_v.at[j]], buf, gsem).wait()
            pltpu.sync_copy(buf, out_hbm.at[pl.ds(base + j * chunk, chunk)])
            return carry

        lax.fori_loop(0, n_ch, body, 0)

    return gather_kernel


def kernel(trg_seq, enc_output, W):
    del enc_output
    batch, seq = trg_seq.shape
    v, d = W.shape
    n_total = batch * seq

    n_workers = 32
    per_w = n_total // n_workers
    chunk = 128
    n_ch = per_w // chunk

    idx = trg_seq.reshape(n_workers, n_ch, chunk).astype(jnp.int32)
    fn = _make_gather(n_workers, per_w, chunk, n_ch, n_total, d)
    out = fn(W, idx)
    return out.reshape(batch, seq, d)


# speedup vs baseline: 1.1147x; 1.1147x over previous
import functools

import jax
import jax.numpy as jnp
from jax import lax
from jax.experimental import pallas as pl
from jax.experimental.pallas import tpu as pltpu
from jax.experimental.pallas import tpu_sc as plsc


_NBUF = 8


def _make_gather(n_workers: int, per_w: int, chunk: int, n_ch: int,
                 n_total: int, d: int):
    mesh = plsc.VectorSubcoreMesh(core_axis_name="c", subcore_axis_name="s")

    @functools.partial(
        pl.kernel,
        mesh=mesh,
        out_type=jax.ShapeDtypeStruct((n_total, d), jnp.float32),
        scratch_types=[
            pltpu.VMEM((n_ch, chunk), jnp.int32),
            pltpu.VMEM((_NBUF, chunk, d), jnp.float32),
            pltpu.SemaphoreType.DMA,
            pltpu.SemaphoreType.DMA,
        ],
        compiler_params=pltpu.CompilerParams(use_tc_tiling_on_sc=False),
    )
    def gather_kernel(table_hbm, idx_hbm, out_hbm, idx_v, bufs, gsem, osem):
        wid = lax.axis_index("s") * 2 + lax.axis_index("c")
        base = wid * per_w
        pltpu.sync_copy(idx_hbm.at[wid], idx_v)

        n_outer = n_ch // _NBUF

        def body(jj, carry):
            j0 = jj * _NBUF
            gh = [
                pltpu.async_copy(table_hbm.at[idx_v.at[j0 + b]],
                                 bufs.at[b], gsem)
                for b in range(_NBUF)
            ]
            oh = []
            for b in range(_NBUF):
                gh[b].wait()
                dst = out_hbm.at[pl.ds(base + (j0 + b) * chunk, chunk)]
                oh.append(pltpu.async_copy(bufs.at[b], dst, osem))
            for b in range(_NBUF):
                oh[b].wait()
            return carry

        lax.fori_loop(0, n_outer, body, 0)

    return gather_kernel


def kernel(trg_seq, enc_output, W):
    del enc_output
    batch, seq = trg_seq.shape
    v, d = W.shape
    n_total = batch * seq

    n_workers = 32
    per_w = n_total // n_workers
    chunk = 128
    n_ch = per_w // chunk

    idx = trg_seq.reshape(n_workers, n_ch, chunk).astype(jnp.int32)
    fn = _make_gather(n_workers, per_w, chunk, n_ch, n_total, d)
    out = fn(W, idx)
    return out.reshape(batch, seq, d)
